# baseline (speedup 1.0000x reference)
import functools

import jax
import jax.numpy as jnp
from jax import lax
from jax.experimental import pallas as pl
from jax.experimental.pallas import tpu as pltpu
from jax.experimental.pallas import tpu_sc as plsc

F32 = jnp.float32
I32 = jnp.int32
NC, NS = 2, 16
NW = NC * NS
SEG = 4096
GQ = 256
CAPL = 24576

_SC_PARAMS = pltpu.CompilerParams(use_tc_tiling_on_sc=False,
                                  needs_layout_passes=False)


def _mesh():
    return plsc.VectorSubcoreMesh(core_axis_name="c", subcore_axis_name="s",
                                  num_cores=NC, num_subcores=NS)


def _rb(n_rows):
    return ((n_rows + NW * 8 - 1) // (NW * 8)) * 8


def _make_sc_partition(n_rows: int, nseg: int, caph: int):
    rb = _rb(n_rows)
    np_rows = rb * NW

    out_type = [jax.ShapeDtypeStruct((NW, caph), I32),
                jax.ShapeDtypeStruct((NW, caph), I32),
                jax.ShapeDtypeStruct((NW, 16), I32),
                jax.ShapeDtypeStruct((np_rows, 8), F32)]
    scratch = [
        pltpu.VMEM((2, 2, SEG), I32),
        pltpu.VMEM((CAPL,), I32),
        pltpu.VMEM((CAPL,), I32),
        pltpu.VMEM((rb + 8, 8), F32),
        pltpu.VMEM((16,), I32),
        pltpu.SemaphoreType.DMA,
        pltpu.SemaphoreType.DMA,
    ]

    def body(edges, zcnt, src_o, rel_o, mlen_o, cnt_o,
             ebuf, cs, cr, cnt2, mbuf, esem0, esem1):
        esem = (esem0, esem1)
        cid = lax.axis_index("c")
        sid = lax.axis_index("s")
        wid = cid * NS + sid
        lo = wid * rb

        pltpu.sync_copy(zcnt, cnt2)

        ones16 = jnp.ones((16,), F32)
        zero16 = jnp.zeros((16,), I32)
        dsrc = jnp.zeros((16,), I32)
        drel = jnp.full((16,), rb, I32)

        def estage(s, b):
            pltpu.async_copy(edges.at[s], ebuf.at[b], esem[b])

        def ewait(s, b):
            pltpu.make_async_copy(edges.at[s], ebuf.at[b], esem[b]).wait()

        def scan_segment(b, m):
            def step(k, m):
                src16 = ebuf[b, 0, pl.ds(k * 16, 16)]
                dst16 = ebuf[b, 1, pl.ds(k * 16, 16)]
                rel16 = dst16 - lo
                mask = plsc.bitcast(rel16, jnp.uint32) < jnp.uint32(rb)
                plsc.store_compressed(cs.at[pl.ds(m, 16)], src16, mask=mask)
                plsc.store_compressed(cr.at[pl.ds(m, 16)], rel16, mask=mask)
                plsc.addupdate_scatter(cnt2, [rel16, zero16], ones16,
                                       mask=mask)
                return m + jnp.sum(mask.astype(I32))

            return lax.fori_loop(0, SEG // 16, step, m)

        def flush(m, mtot):
            for k in range(GQ // 16):
                cs[pl.ds(m + k * 16, 16)] = dsrc
                cr[pl.ds(m + k * 16, 16)] = drel
            mq = (m + GQ - 1) & ~(GQ - 1)

            def fstep(f, carry):
                off = pl.multiple_of(mtot + f * GQ, GQ)
                pltpu.sync_copy(cs.at[pl.ds(f * GQ, GQ)],
                                src_o.at[wid, pl.ds(off, GQ)])
                pltpu.sync_copy(cr.at[pl.ds(f * GQ, GQ)],
                                rel_o.at[wid, pl.ds(off, GQ)])
                return carry

            lax.fori_loop(0, mq // GQ, fstep, 0)
            return mtot + mq

        def dummy_block(mtot):
            for k in range(GQ // 16):
                cs[pl.ds(k * 16, 16)] = dsrc
                cr[pl.ds(k * 16, 16)] = drel
            offd = pl.multiple_of(mtot, GQ)
            pltpu.sync_copy(cs.at[pl.ds(0, GQ)],
                            src_o.at[wid, pl.ds(offd, GQ)])
            pltpu.sync_copy(cr.at[pl.ds(0, GQ)],
                            rel_o.at[wid, pl.ds(offd, GQ)])
            return mtot + GQ

        estage(0, 0)

        def seg_pair(g, carry):
            m, mtot = carry
            for b in (0, 1):
                s = 2 * g + b
                ewait(s, b)
                estage(s + 1, 1 - b)
                m = scan_segment(b, m)
                m, mtot = lax.cond(
                    m >= CAPL - SEG - GQ,
                    lambda mm, mt: (jnp.int32(0), flush(mm, mt)),
                    lambda mm, mt: (mm, mt),
                    m, mtot)
            return m, mtot

        m, mtot = lax.fori_loop(0, nseg // 2, seg_pair,
                                (jnp.int32(0), jnp.int32(0)))
        ewait(nseg, 0)
        mtot = flush(m, mtot)
        mtot = lax.cond((mtot // GQ) % 2 == 1, dummy_block,
                        lambda mt: mt, mtot)
        dummy_block(mtot)

        mbuf[...] = jnp.full((16,), mtot, I32)
        pltpu.sync_copy(mbuf, mlen_o.at[wid])
        pltpu.sync_copy(cnt2.at[pl.ds(0, rb)], cnt_o.at[pl.ds(wid * rb, rb)])

    return pl.kernel(body, out_type=out_type, mesh=_mesh(),
                     scratch_types=scratch, compiler_params=_SC_PARAMS)


def _make_sc_aggregate(n_rows: int, d: int, caph: int):
    rb = _rb(n_rows)
    np_rows = rb * NW

    out_type = [jax.ShapeDtypeStruct((np_rows, d), F32)]
    scratch = [
        pltpu.VMEM((rb + 8, d), F32),
        pltpu.VMEM((2, GQ), I32),
        pltpu.VMEM((2, GQ), I32),
        pltpu.VMEM((GQ, d), F32),
        pltpu.VMEM((GQ, d), F32),
        pltpu.VMEM((16,), I32),
        pltpu.SemaphoreType.DMA,
        pltpu.SemaphoreType.DMA,
    ]

    def body(table, src_i, rel_i, mlen_i, zrow, acc_o,
             acc_l, cbs, cbr, rows0, rows1, mbuf, gsem0, gsem1):
        rows = (rows0, rows1)
        gsem = (gsem0, gsem1)
        cid = lax.axis_index("c")
        sid = lax.axis_index("s")
        wid = cid * NS + sid

        pltpu.sync_copy(zrow, acc_l)
        pltpu.sync_copy(mlen_i.at[wid], mbuf)
        nblk = mbuf[...][0] // GQ

        def stage(f, b):
            off = pl.multiple_of(f * GQ, GQ)
            pltpu.sync_copy(src_i.at[wid, pl.ds(off, GQ)], cbs.at[b])
            pltpu.sync_copy(rel_i.at[wid, pl.ds(off, GQ)], cbr.at[b])
            pltpu.async_copy(table.at[cbs.at[b]], rows[b], gsem[b])

        def gwait(b):
            pltpu.make_async_copy(table.at[cbs.at[b]], rows[b],
                                  gsem[b]).wait()

        def accum(b):
            def astep(kv, carry):
                rv = cbr[b, pl.ds(kv * 16, 16)]
                for q in range(16):
                    r = rv[q]
                    i = kv * 16 + q
                    for c in range(d // 16):
                        sl = pl.ds(c * 16, 16)
                        acc_l[r, sl] = acc_l[r, sl] + rows[b][i, sl]
                return carry

            lax.fori_loop(0, GQ // 16, astep, 0)

        stage(0, 0)

        def pair(g, carry):
            for b in (0, 1):
                f = 2 * g + b
                gwait(b)
                stage(f + 1, 1 - b)
                accum(b)
            return carry

        lax.fori_loop(0, nblk // 2, pair, 0)
        gwait(0)
        pltpu.sync_copy(acc_l.at[pl.ds(0, rb)],
                        acc_o.at[pl.ds(wid * rb, rb)])

    return pl.kernel(body, out_type=out_type, mesh=_mesh(),
                     scratch_types=scratch, compiler_params=_SC_PARAMS)


def _tc_mid(x, acc, cnt, w1l, b1, w1r, w2l, *, bn):
    n, f = x.shape
    h_dim = w1l.shape[1]
    p_dim = w2l.shape[1]
    grid = (n // bn,)

    def body(x_r, a_r, c_r, w1l_r, b1_r, w1r_r, w2l_r, h_r, p_r):
        deg = jnp.sum(c_r[...], axis=1, keepdims=True)
        recip = 1.0 / jnp.maximum(deg, 1.0)
        mean = a_r[...] * recip
        h = jnp.dot(mean, w1l_r[...], preferred_element_type=F32) + b1_r[...]
        h = h + jnp.dot(x_r[...], w1r_r[...], preferred_element_type=F32)
        h = jnp.maximum(h, 0.0)
        h_r[...] = h
        p_r[...] = jnp.dot(h, w2l_r[...], preferred_element_type=F32)

    return pl.pallas_call(
        body,
        grid=grid,
        in_specs=[
            pl.BlockSpec((bn, f), lambda i: (i, 0)),
            pl.BlockSpec((bn, f), lambda i: (i, 0)),
            pl.BlockSpec((bn, 8), lambda i: (i, 0)),
            pl.BlockSpec((f, h_dim), lambda i: (0, 0)),
            pl.BlockSpec((1, h_dim), lambda i: (0, 0)),
            pl.BlockSpec((f, h_dim), lambda i: (0, 0)),
            pl.BlockSpec((h_dim, p_dim), lambda i: (0, 0)),
        ],
        out_specs=[
            pl.BlockSpec((bn, h_dim), lambda i: (i, 0)),
            pl.BlockSpec((bn, p_dim), lambda i: (i, 0)),
        ],
        out_shape=[jax.ShapeDtypeStruct((n, h_dim), F32),
                   jax.ShapeDtypeStruct((n, p_dim), F32)],
    )(x, acc, cnt, w1l, b1, w1r, w2l)


def _tc_out(h, pacc, cnt, w2r, b2, *, bn):
    n, h_dim = h.shape
    p_dim = w2r.shape[1]
    grid = (n // bn,)

    def body(h_r, p_r, c_r, w2r_r, b2_r, o_r):
        deg = jnp.sum(c_r[...], axis=1, keepdims=True)
        recip = 1.0 / jnp.maximum(deg, 1.0)
        meanp = p_r[...] * recip
        o_r[...] = meanp + b2_r[...] + jnp.dot(
            h_r[...], w2r_r[...], preferred_element_type=F32)

    return pl.pallas_call(
        body,
        grid=grid,
        in_specs=[
            pl.BlockSpec((bn, h_dim), lambda i: (i, 0)),
            pl.BlockSpec((bn, p_dim), lambda i: (i, 0)),
            pl.BlockSpec((bn, 8), lambda i: (i, 0)),
            pl.BlockSpec((h_dim, p_dim), lambda i: (0, 0)),
            pl.BlockSpec((1, p_dim), lambda i: (0, 0)),
        ],
        out_specs=pl.BlockSpec((bn, p_dim), lambda i: (i, 0)),
        out_shape=jax.ShapeDtypeStruct((n, p_dim), F32),
    )(h, pacc, cnt, w2r, b2)


def kernel(x, edge_index, W1l, b1, W1r, W2l, b2, W2r):
    n, f = x.shape
    e = edge_index.shape[1]
    o = W2l.shape[1]
    p_dim = 128
    bn = 400
    rb = _rb(n)

    nseg = -(-e // SEG)
    nseg += nseg % 2
    e_pad = nseg * SEG
    src = jnp.concatenate(
        [edge_index[0], jnp.zeros((e_pad - e,), I32)]).reshape(nseg, 1, SEG)
    dst = jnp.concatenate(
        [edge_index[1], jnp.full((e_pad - e,), n, I32)]).reshape(nseg, 1, SEG)
    pad_seg = jnp.concatenate(
        [jnp.zeros((1, 1, SEG), I32), jnp.full((1, 1, SEG), n, I32)], axis=1)
    edges = jnp.concatenate(
        [jnp.concatenate([src, dst], axis=1), pad_seg], axis=0)

    caph = e_pad + 8192
    zcnt = jnp.zeros((rb + 8, 8), F32)
    zrow = jnp.zeros((rb + 8, f), F32)

    part = _make_sc_partition(n, nseg, caph)
    csrc, crel, mlen, cnt = part(edges, zcnt)

    agg = _make_sc_aggregate(n, f, caph)
    (acc,) = agg(x, csrc, crel, mlen, zrow)

    w2l_p = jnp.pad(W2l, ((0, 0), (0, p_dim - o)))
    h, p = _tc_mid(x, acc, cnt, W1l, b1.reshape(1, -1), W1r, w2l_p, bn=bn)

    agg2 = _make_sc_aggregate(n, p_dim, caph)
    (pacc,) = agg2(p, csrc, crel, mlen, zrow)

    w2r_p = jnp.pad(W2r, ((0, 0), (0, p_dim - o)))
    b2_p = jnp.pad(b2, (0, p_dim - o)).reshape(1, -1)
    out = _tc_out(h, pacc, cnt, w2r_p, b2_p, bn=bn)
    return out[:, :o]


# speedup vs baseline: 2.0025x; 2.0025x over previous
import functools

import jax
import jax.numpy as jnp
from jax import lax
from jax.experimental import pallas as pl
from jax.experimental.pallas import tpu as pltpu
from jax.experimental.pallas import tpu_sc as plsc

F32 = jnp.float32
NC, NS = 2, 16
NW = NC * NS
CH = 144
CNTW = 16


def _make_sc_agg(n_rows: int, d: int, nchunk: int, ch: int, with_count: bool):
    assert nchunk % 4 == 0 and nchunk >= 8
    assert not with_count or ch % 16 == 0
    rpt = ((n_rows + NS * 8 - 1) // (NS * 8)) * 8
    np_rows = rpt * NS

    mesh = plsc.VectorSubcoreMesh(core_axis_name="c", subcore_axis_name="s",
                                  num_cores=NC, num_subcores=NS)

    out_type = [jax.ShapeDtypeStruct((np_rows, d), F32),
                jax.ShapeDtypeStruct((np_rows, d), F32)]
    scratch = [
        pltpu.VMEM((4, 2, ch), jnp.int32),
        pltpu.VMEM((ch, d), F32),
        pltpu.VMEM((ch, d), F32),
        pltpu.VMEM((ch, d), F32),
        pltpu.VMEM((ch, d), F32),
        pltpu.VMEM_SHARED((np_rows, d), F32),
    ] + [pltpu.SemaphoreType.DMA] * 12
    if with_count:
        out_type += [jax.ShapeDtypeStruct((NW, np_rows), F32)]
        scratch += [pltpu.VMEM((np_rows,), F32)]

    def body(*refs):
        if with_count:
            (table, edges, zrow,
             acc0_o, acc1_o, cnt_o,
             idx, rows0, rows1, rows2, rows3, acc_sh, *sems) = refs
            cnt_l = sems[-1]
            sems = sems[:-1]
        else:
            (table, edges, zrow,
             acc0_o, acc1_o,
             idx, rows0, rows1, rows2, rows3, acc_sh, *sems) = refs
        rows = (rows0, rows1, rows2, rows3)
        isem, gsem, ssem = sems[0:4], sems[4:8], sems[8:12]
        cid = lax.axis_index("c")
        sid = lax.axis_index("s")
        wid = cid * NS + sid

        pltpu.sync_copy(zrow, acc_sh.at[pl.ds(sid * rpt, rpt)])
        if with_count:
            zv = jnp.zeros((16,), F32)

            def zstep(i, carry):
                cnt_l[pl.ds(i * 16, 16)] = zv
                return carry

            lax.fori_loop(0, np_rows // 16, zstep, 0)
        plsc.subcore_barrier()

        ones16 = jnp.ones((16,), F32)

        def istart(j, s):
            pltpu.async_copy(edges.at[wid, j], idx.at[s], isem[s])

        def iwait(j, s):
            pltpu.make_async_copy(edges.at[wid, j], idx.at[s], isem[s]).wait()

        def gstart(s):
            pltpu.async_copy(table.at[idx.at[s, 0]], rows[s], gsem[s])

        def gwait(s):
            pltpu.make_async_copy(table.at[idx.at[s, 0]], rows[s],
                                  gsem[s]).wait()

        def sstart(s):
            pltpu.async_copy(rows[s], acc_sh.at[idx.at[s, 1]], ssem[s],
                             add=True)

        def swait(s):
            pltpu.make_async_copy(rows[s], acc_sh.at[idx.at[s, 1]],
                                  ssem[s]).wait()

        def counts(s):
            if with_count:
                for k in range(ch // 16):
                    dv = idx[s, 1, pl.ds(k * 16, 16)]
                    plsc.addupdate_scatter(cnt_l, [dv], ones16)

        istart(0, 0)
        istart(1, 1)
        iwait(0, 0)
        gstart(0)
        istart(2, 2)
        iwait(1, 1)
        gstart(1)
        istart(3, 3)
        gwait(0)
        counts(0)
        sstart(0)

        def group(g, carry):
            for k in range(4):
                j = 4 * g + 2 + k
                b, b1, b2 = (2 + k) % 4, (1 + k) % 4, k
                iwait(j, b)
                gstart(b)
                gwait(b1)
                counts(b1)
                sstart(b1)
                swait(b2)
                istart(j + 2, b2)
            return carry

        lax.fori_loop(0, (nchunk - 4) // 4, group, 0)
        iwait(nchunk - 2, 2)
        gstart(2)
        gwait(1)
        counts(1)
        sstart(1)
        swait(0)
        iwait(nchunk - 1, 3)
        gstart(3)
        gwait(2)
        counts(2)
        sstart(2)
        swait(1)
        gwait(3)
        counts(3)
        sstart(3)
        swait(2)
        swait(3)
        plsc.subcore_barrier()

        sl = pl.ds(sid * rpt, rpt)
        if with_count:
            pltpu.sync_copy(cnt_l, cnt_o.at[wid])

        @pl.when(cid == 0)
        def _():
            pltpu.sync_copy(acc_sh.at[sl], acc0_o.at[sl])

        @pl.when(cid == 1)
        def _():
            pltpu.sync_copy(acc_sh.at[sl], acc1_o.at[sl])

    return pl.kernel(body, out_type=out_type, mesh=mesh, scratch_types=scratch,
                     compiler_params=pltpu.CompilerParams(
                         use_tc_tiling_on_sc=False, needs_layout_passes=False))


def _tc_mid(x, acc0, acc1, cntT, w1l, b1, w1r, w2l, *, bn):
    n, f = x.shape
    h_dim = w1l.shape[1]
    p_dim = w2l.shape[1]
    grid = (n // bn,)

    def body(x_r, a0_r, a1_r, c_r, w1l_r, b1_r, w1r_r, w2l_r, h_r, p_r):
        cnt = jnp.sum(c_r[...], axis=1, keepdims=True)
        recip = 1.0 / jnp.maximum(cnt, 1.0)
        mean = (a0_r[...] + a1_r[...]) * recip
        h = jnp.dot(mean, w1l_r[...], preferred_element_type=F32) + b1_r[...]
        h = h + jnp.dot(x_r[...], w1r_r[...], preferred_element_type=F32)
        h = jnp.maximum(h, 0.0)
        h_r[...] = h
        p_r[...] = jnp.dot(h, w2l_r[...], preferred_element_type=F32)

    return pl.pallas_call(
        body,
        grid=grid,
        in_specs=[
            pl.BlockSpec((bn, f), lambda i: (i, 0)),
            pl.BlockSpec((bn, f), lambda i: (i, 0)),
            pl.BlockSpec((bn, f), lambda i: (i, 0)),
            pl.BlockSpec((bn, NW), lambda i: (i, 0)),
            pl.BlockSpec((f, h_dim), lambda i: (0, 0)),
            pl.BlockSpec((1, h_dim), lambda i: (0, 0)),
            pl.BlockSpec((f, h_dim), lambda i: (0, 0)),
            pl.BlockSpec((h_dim, p_dim), lambda i: (0, 0)),
        ],
        out_specs=[
            pl.BlockSpec((bn, h_dim), lambda i: (i, 0)),
            pl.BlockSpec((bn, p_dim), lambda i: (i, 0)),
        ],
        out_shape=[jax.ShapeDtypeStruct((n, h_dim), F32),
                   jax.ShapeDtypeStruct((n, p_dim), F32)],
    )(x, acc0, acc1, cntT, w1l, b1, w1r, w2l)


def _tc_out(h, p0, p1, cntT, w2r, b2, *, bn):
    n, h_dim = h.shape
    p_dim = w2r.shape[1]
    grid = (n // bn,)

    def body(h_r, p0_r, p1_r, c_r, w2r_r, b2_r, o_r):
        cnt = jnp.sum(c_r[...], axis=1, keepdims=True)
        recip = 1.0 / jnp.maximum(cnt, 1.0)
        meanp = (p0_r[...] + p1_r[...]) * recip
        o_r[...] = meanp + b2_r[...] + jnp.dot(
            h_r[...], w2r_r[...], preferred_element_type=F32)

    return pl.pallas_call(
        body,
        grid=grid,
        in_specs=[
            pl.BlockSpec((bn, h_dim), lambda i: (i, 0)),
            pl.BlockSpec((bn, p_dim), lambda i: (i, 0)),
            pl.BlockSpec((bn, p_dim), lambda i: (i, 0)),
            pl.BlockSpec((bn, NW), lambda i: (i, 0)),
            pl.BlockSpec((h_dim, p_dim), lambda i: (0, 0)),
            pl.BlockSpec((1, p_dim), lambda i: (0, 0)),
        ],
        out_specs=pl.BlockSpec((bn, p_dim), lambda i: (i, 0)),
        out_shape=jax.ShapeDtypeStruct((n, p_dim), F32),
    )(h, p0, p1, cntT, w2r, b2)


def kernel(x, edge_index, W1l, b1, W1r, W2l, b2, W2r):
    n, f = x.shape
    e = edge_index.shape[1]
    o = W2l.shape[1]
    p_dim = 128
    bn = 400

    def edges_for(ch):
        nchunk = -(-e // (NW * ch))
        nchunk = ((nchunk + 3) // 4) * 4
        e_pad = nchunk * ch * NW
        src = jnp.concatenate(
            [edge_index[0], jnp.zeros((e_pad - e,), jnp.int32)]).reshape(
                NW, nchunk, 1, ch)
        dst = jnp.concatenate(
            [edge_index[1], jnp.full((e_pad - e,), n, jnp.int32)]).reshape(
                NW, nchunk, 1, ch)
        return jnp.concatenate([src, dst], axis=2), nchunk

    ch1, ch2 = 64, 96
    edges1, nchunk1 = edges_for(ch1)
    edges2, nchunk2 = edges_for(ch2)

    rpt = ((n + NS * 8 - 1) // (NS * 8)) * 8
    zrow = jnp.zeros((rpt, f), F32)

    agg1 = _make_sc_agg(n, f, nchunk1, ch1, with_count=True)
    acc0, acc1, cnt_part = agg1(x, edges1, zrow)
    cntT = cnt_part.T

    w2l_p = jnp.pad(W2l, ((0, 0), (0, p_dim - o)))
    h, p = _tc_mid(x, acc0, acc1, cntT,
                   W1l, b1.reshape(1, -1), W1r, w2l_p, bn=bn)

    agg2 = _make_sc_agg(n, p_dim, nchunk2, ch2, with_count=False)
    pacc0, pacc1 = agg2(p, edges2, zrow)

    w2r_p = jnp.pad(W2r, ((0, 0), (0, p_dim - o)))
    b2_p = jnp.pad(b2, (0, p_dim - o)).reshape(1, -1)
    out = _tc_out(h, pacc0, pacc1, cntT, w2r_p, b2_p, bn=bn)
    return out[:, :o]


# speedup vs baseline: 2.3797x; 1.1883x over previous
import functools

import jax
import jax.numpy as jnp
from jax import lax
from jax.experimental import pallas as pl
from jax.experimental.pallas import tpu as pltpu
from jax.experimental.pallas import tpu_sc as plsc

F32 = jnp.float32
NC, NS = 2, 16
NW = NC * NS
CH = 144
CNTW = 16


def _make_sc_agg(n_rows: int, d: int, nchunk: int, ch: int, with_count: bool):
    assert nchunk % 2 == 0
    assert not with_count or ch % 16 == 0
    rpt = ((n_rows + NS * 8 - 1) // (NS * 8)) * 8
    np_rows = rpt * NS

    mesh = plsc.VectorSubcoreMesh(core_axis_name="c", subcore_axis_name="s",
                                  num_cores=NC, num_subcores=NS)

    out_type = [jax.ShapeDtypeStruct((np_rows, d), F32),
                jax.ShapeDtypeStruct((np_rows, d), F32)]
    scratch = [
        pltpu.VMEM((2, 2, ch), jnp.int32),
        pltpu.VMEM((ch, d), F32),
        pltpu.VMEM((ch, d), F32),
        pltpu.VMEM_SHARED((np_rows, d), F32),
        pltpu.SemaphoreType.DMA,
        pltpu.SemaphoreType.DMA,
        pltpu.SemaphoreType.DMA,
        pltpu.SemaphoreType.DMA,
    ]
    if with_count:
        out_type += [jax.ShapeDtypeStruct((NW, np_rows), F32)]
        scratch += [pltpu.VMEM((np_rows,), F32)]

    def body(*refs):
        if with_count:
            (table, edges, zrow,
             acc0_o, acc1_o, cnt_o,
             idx, rows0, rows1, acc_sh, gsem0, gsem1, ssem0, ssem1,
             cnt_l) = refs
        else:
            (table, edges, zrow,
             acc0_o, acc1_o,
             idx, rows0, rows1, acc_sh, gsem0, gsem1, ssem0, ssem1) = refs
        rows = (rows0, rows1)
        gsem = (gsem0, gsem1)
        ssem = (ssem0, ssem1)
        cid = lax.axis_index("c")
        sid = lax.axis_index("s")
        wid = cid * NS + sid

        pltpu.sync_copy(zrow, acc_sh.at[pl.ds(sid * rpt, rpt)])
        if with_count:
            zv = jnp.zeros((16,), F32)

            def zstep(i, carry):
                cnt_l[pl.ds(i * 16, 16)] = zv
                return carry

            lax.fori_loop(0, np_rows // 16, zstep, 0)
        plsc.subcore_barrier()

        ones16 = jnp.ones((16,), F32)

        def stage(j, b):
            pltpu.sync_copy(edges.at[wid, j], idx.at[b])
            pltpu.async_copy(table.at[idx.at[b, 0]], rows[b], gsem[b])

        def gwait(b):
            pltpu.make_async_copy(table.at[idx.at[b, 0]], rows[b],
                                  gsem[b]).wait()

        def sstart(b):
            pltpu.async_copy(rows[b], acc_sh.at[idx.at[b, 1]], ssem[b],
                             add=True)

        def swait(b):
            pltpu.make_async_copy(rows[b], acc_sh.at[idx.at[b, 1]],
                                  ssem[b]).wait()

        def counts(b):
            if with_count:
                for k in range(ch // 16):
                    dv = idx[b, 1, pl.ds(k * 16, 16)]
                    plsc.addupdate_scatter(cnt_l, [dv], ones16)

        stage(0, 0)
        gwait(0)
        counts(0)
        sstart(0)
        stage(1, 1)

        def pair(g, carry):
            for b in (1, 0):
                j = 2 * g + 2 - b
                nb = 1 - b
                gwait(b)
                counts(b)
                sstart(b)
                swait(nb)
                stage(j + 1, nb)
            return carry

        lax.fori_loop(0, (nchunk - 2) // 2, pair, 0)
        gwait(1)
        counts(1)
        sstart(1)
        swait(0)
        swait(1)
        plsc.subcore_barrier()

        sl = pl.ds(sid * rpt, rpt)
        if with_count:
            pltpu.sync_copy(cnt_l, cnt_o.at[wid])

        @pl.when(cid == 0)
        def _():
            pltpu.sync_copy(acc_sh.at[sl], acc0_o.at[sl])

        @pl.when(cid == 1)
        def _():
            pltpu.sync_copy(acc_sh.at[sl], acc1_o.at[sl])

    return pl.kernel(body, out_type=out_type, mesh=mesh, scratch_types=scratch,
                     compiler_params=pltpu.CompilerParams(
                         use_tc_tiling_on_sc=False, needs_layout_passes=False))


def _tc_mid(x, acc0, acc1, cntT, w1l, b1, w1r, w2l, *, bn):
    n, f = x.shape
    h_dim = w1l.shape[1]
    p_dim = w2l.shape[1]
    grid = (n // bn,)

    def body(x_r, a0_r, a1_r, c_r, w1l_r, b1_r, w1r_r, w2l_r, h_r, p_r):
        cnt = jnp.sum(c_r[...], axis=1, keepdims=True)
        recip = 1.0 / jnp.maximum(cnt, 1.0)
        mean = (a0_r[...] + a1_r[...]) * recip
        h = jnp.dot(mean, w1l_r[...], preferred_element_type=F32) + b1_r[...]
        h = h + jnp.dot(x_r[...], w1r_r[...], preferred_element_type=F32)
        h = jnp.maximum(h, 0.0)
        h_r[...] = h
        p_r[...] = jnp.dot(h, w2l_r[...], preferred_element_type=F32)

    return pl.pallas_call(
        body,
        grid=grid,
        in_specs=[
            pl.BlockSpec((bn, f), lambda i: (i, 0)),
            pl.BlockSpec((bn, f), lambda i: (i, 0)),
            pl.BlockSpec((bn, f), lambda i: (i, 0)),
            pl.BlockSpec((bn, NW), lambda i: (i, 0)),
            pl.BlockSpec((f, h_dim), lambda i: (0, 0)),
            pl.BlockSpec((1, h_dim), lambda i: (0, 0)),
            pl.BlockSpec((f, h_dim), lambda i: (0, 0)),
            pl.BlockSpec((h_dim, p_dim), lambda i: (0, 0)),
        ],
        out_specs=[
            pl.BlockSpec((bn, h_dim), lambda i: (i, 0)),
            pl.BlockSpec((bn, p_dim), lambda i: (i, 0)),
        ],
        out_shape=[jax.ShapeDtypeStruct((n, h_dim), F32),
                   jax.ShapeDtypeStruct((n, p_dim), F32)],
    )(x, acc0, acc1, cntT, w1l, b1, w1r, w2l)


def _tc_out(h, p0, p1, cntT, w2r, b2, *, bn):
    n, h_dim = h.shape
    p_dim = w2r.shape[1]
    grid = (n // bn,)

    def body(h_r, p0_r, p1_r, c_r, w2r_r, b2_r, o_r):
        cnt = jnp.sum(c_r[...], axis=1, keepdims=True)
        recip = 1.0 / jnp.maximum(cnt, 1.0)
        meanp = (p0_r[...] + p1_r[...]) * recip
        o_r[...] = meanp + b2_r[...] + jnp.dot(
            h_r[...], w2r_r[...], preferred_element_type=F32)

    return pl.pallas_call(
        body,
        grid=grid,
        in_specs=[
            pl.BlockSpec((bn, h_dim), lambda i: (i, 0)),
            pl.BlockSpec((bn, p_dim), lambda i: (i, 0)),
            pl.BlockSpec((bn, p_dim), lambda i: (i, 0)),
            pl.BlockSpec((bn, NW), lambda i: (i, 0)),
            pl.BlockSpec((h_dim, p_dim), lambda i: (0, 0)),
            pl.BlockSpec((1, p_dim), lambda i: (0, 0)),
        ],
        out_specs=pl.BlockSpec((bn, p_dim), lambda i: (i, 0)),
        out_shape=jax.ShapeDtypeStruct((n, p_dim), F32),
    )(h, p0, p1, cntT, w2r, b2)


def kernel(x, edge_index, W1l, b1, W1r, W2l, b2, W2r):
    n, f = x.shape
    e = edge_index.shape[1]
    o = W2l.shape[1]
    p_dim = 128
    bn = 400

    def edges_for(ch):
        nchunk = -(-e // (NW * ch))
        nchunk += nchunk % 2
        e_pad = nchunk * ch * NW
        src = jnp.concatenate(
            [edge_index[0], jnp.zeros((e_pad - e,), jnp.int32)]).reshape(
                NW, nchunk, 1, ch)
        dst = jnp.concatenate(
            [edge_index[1], jnp.full((e_pad - e,), n, jnp.int32)]).reshape(
                NW, nchunk, 1, ch)
        pad_chunk = jnp.concatenate(
            [jnp.zeros((NW, 1, 1, ch), jnp.int32),
             jnp.full((NW, 1, 1, ch), n, jnp.int32)], axis=2)
        return jnp.concatenate(
            [jnp.concatenate([src, dst], axis=2), pad_chunk], axis=1), nchunk

    ch1, ch2 = 144, 192
    edges1, nchunk1 = edges_for(ch1)
    edges2, nchunk2 = edges_for(ch2)

    rpt = ((n + NS * 8 - 1) // (NS * 8)) * 8
    zrow = jnp.zeros((rpt, f), F32)

    agg1 = _make_sc_agg(n, f, nchunk1, ch1, with_count=True)
    acc0, acc1, cnt_part = agg1(x, edges1, zrow)
    cntT = cnt_part.T

    w2l_p = jnp.pad(W2l, ((0, 0), (0, p_dim - o)))
    h, p = _tc_mid(x, acc0, acc1, cntT,
                   W1l, b1.reshape(1, -1), W1r, w2l_p, bn=bn)

    agg2 = _make_sc_agg(n, p_dim, nchunk2, ch2, with_count=False)
    pacc0, pacc1 = agg2(p, edges2, zrow)

    w2r_p = jnp.pad(W2r, ((0, 0), (0, p_dim - o)))
    b2_p = jnp.pad(b2, (0, p_dim - o)).reshape(1, -1)
    out = _tc_out(h, pacc0, pacc1, cntT, w2r_p, b2_p, bn=bn)
    return out[:, :o]


# speedup vs baseline: 3.9178x; 1.6463x over previous
import functools

import jax
import jax.numpy as jnp
from jax import lax
from jax.experimental import pallas as pl
from jax.experimental.pallas import tpu as pltpu
from jax.experimental.pallas import tpu_sc as plsc

F32 = jnp.float32
NC, NS = 2, 16
NW = NC * NS
CH = 144
CNTW = 16


def _make_sc_agg(n_rows: int, d: int, nchunk: int, with_count: bool):
    assert nchunk % 2 == 0
    rpt = ((n_rows + NS * 8 - 1) // (NS * 8)) * 8
    np_rows = rpt * NS

    mesh = plsc.VectorSubcoreMesh(core_axis_name="c", subcore_axis_name="s",
                                  num_cores=NC, num_subcores=NS)

    out_type = [jax.ShapeDtypeStruct((np_rows, d), F32),
                jax.ShapeDtypeStruct((np_rows, d), F32)]
    scratch = [
        pltpu.VMEM((2, 2, CH), jnp.int32),
        pltpu.VMEM((CH, d), F32),
        pltpu.VMEM((CH, d), F32),
        pltpu.VMEM_SHARED((np_rows, d), F32),
        pltpu.SemaphoreType.DMA,
        pltpu.SemaphoreType.DMA,
        pltpu.SemaphoreType.DMA,
        pltpu.SemaphoreType.DMA,
    ]
    if with_count:
        out_type += [jax.ShapeDtypeStruct((NW, np_rows), F32)]
        scratch += [pltpu.VMEM((np_rows,), F32)]

    def body(*refs):
        if with_count:
            (table, edges, zrow,
             acc0_o, acc1_o, cnt_o,
             idx, rows0, rows1, acc_sh, gsem0, gsem1, ssem0, ssem1,
             cnt_l) = refs
        else:
            (table, edges, zrow,
             acc0_o, acc1_o,
             idx, rows0, rows1, acc_sh, gsem0, gsem1, ssem0, ssem1) = refs
        rows = (rows0, rows1)
        gsem = (gsem0, gsem1)
        ssem = (ssem0, ssem1)
        cid = lax.axis_index("c")
        sid = lax.axis_index("s")
        wid = cid * NS + sid

        pltpu.sync_copy(zrow, acc_sh.at[pl.ds(sid * rpt, rpt)])
        if with_count:
            zv = jnp.zeros((16,), F32)

            def zstep(i, carry):
                cnt_l[pl.ds(i * 16, 16)] = zv
                return carry

            lax.fori_loop(0, np_rows // 16, zstep, 0)
        plsc.subcore_barrier()

        ones16 = jnp.ones((16,), F32)

        def stage(j, b):
            pltpu.sync_copy(edges.at[wid, j], idx.at[b])
            pltpu.async_copy(table.at[idx.at[b, 0]], rows[b], gsem[b])

        def gwait(b):
            pltpu.make_async_copy(table.at[idx.at[b, 0]], rows[b],
                                  gsem[b]).wait()

        def sstart(b):
            pltpu.async_copy(rows[b], acc_sh.at[idx.at[b, 1]], ssem[b],
                             add=True)

        def swait(b):
            pltpu.make_async_copy(rows[b], acc_sh.at[idx.at[b, 1]],
                                  ssem[b]).wait()

        def counts(b):
            if with_count:
                for k in range(CH // 16):
                    dv = idx[b, 1, pl.ds(k * 16, 16)]
                    plsc.addupdate_scatter(cnt_l, [dv], ones16)

        stage(0, 0)
        gwait(0)
        counts(0)
        sstart(0)
        stage(1, 1)

        def pair(g, carry):
            for b in (1, 0):
                j = 2 * g + 2 - b
                nb = 1 - b
                gwait(b)
                counts(b)
                sstart(b)
                swait(nb)
                stage(j + 1, nb)
            return carry

        lax.fori_loop(0, (nchunk - 2) // 2, pair, 0)
        gwait(1)
        counts(1)
        sstart(1)
        swait(0)
        swait(1)
        plsc.subcore_barrier()

        sl = pl.ds(sid * rpt, rpt)
        if with_count:
            pltpu.sync_copy(cnt_l, cnt_o.at[wid])

        @pl.when(cid == 0)
        def _():
            pltpu.sync_copy(acc_sh.at[sl], acc0_o.at[sl])

        @pl.when(cid == 1)
        def _():
            pltpu.sync_copy(acc_sh.at[sl], acc1_o.at[sl])

    return pl.kernel(body, out_type=out_type, mesh=mesh, scratch_types=scratch,
                     compiler_params=pltpu.CompilerParams(
                         use_tc_tiling_on_sc=False, needs_layout_passes=False))


def _tc_mid(x, acc0, acc1, cntT, w1l, b1, w1r, w2l, *, bn):
    n, f = x.shape
    h_dim = w1l.shape[1]
    p_dim = w2l.shape[1]
    grid = (n // bn,)

    def body(x_r, a0_r, a1_r, c_r, w1l_r, b1_r, w1r_r, w2l_r, h_r, p_r):
        cnt = jnp.sum(c_r[...], axis=1, keepdims=True)
        recip = 1.0 / jnp.maximum(cnt, 1.0)
        mean = (a0_r[...] + a1_r[...]) * recip
        h = jnp.dot(mean, w1l_r[...], preferred_element_type=F32) + b1_r[...]
        h = h + jnp.dot(x_r[...], w1r_r[...], preferred_element_type=F32)
        h = jnp.maximum(h, 0.0)
        h_r[...] = h
        p_r[...] = jnp.dot(h, w2l_r[...], preferred_element_type=F32)

    return pl.pallas_call(
        body,
        grid=grid,
        in_specs=[
            pl.BlockSpec((bn, f), lambda i: (i, 0)),
            pl.BlockSpec((bn, f), lambda i: (i, 0)),
            pl.BlockSpec((bn, f), lambda i: (i, 0)),
            pl.BlockSpec((bn, NW), lambda i: (i, 0)),
            pl.BlockSpec((f, h_dim), lambda i: (0, 0)),
            pl.BlockSpec((1, h_dim), lambda i: (0, 0)),
            pl.BlockSpec((f, h_dim), lambda i: (0, 0)),
            pl.BlockSpec((h_dim, p_dim), lambda i: (0, 0)),
        ],
        out_specs=[
            pl.BlockSpec((bn, h_dim), lambda i: (i, 0)),
            pl.BlockSpec((bn, p_dim), lambda i: (i, 0)),
        ],
        out_shape=[jax.ShapeDtypeStruct((n, h_dim), F32),
                   jax.ShapeDtypeStruct((n, p_dim), F32)],
    )(x, acc0, acc1, cntT, w1l, b1, w1r, w2l)


def _tc_out(h, p0, p1, cntT, w2r, b2, *, bn):
    n, h_dim = h.shape
    p_dim = w2r.shape[1]
    grid = (n // bn,)

    def body(h_r, p0_r, p1_r, c_r, w2r_r, b2_r, o_r):
        cnt = jnp.sum(c_r[...], axis=1, keepdims=True)
        recip = 1.0 / jnp.maximum(cnt, 1.0)
        meanp = (p0_r[...] + p1_r[...]) * recip
        o_r[...] = meanp + b2_r[...] + jnp.dot(
            h_r[...], w2r_r[...], preferred_element_type=F32)

    return pl.pallas_call(
        body,
        grid=grid,
        in_specs=[
            pl.BlockSpec((bn, h_dim), lambda i: (i, 0)),
            pl.BlockSpec((bn, p_dim), lambda i: (i, 0)),
            pl.BlockSpec((bn, p_dim), lambda i: (i, 0)),
            pl.BlockSpec((bn, NW), lambda i: (i, 0)),
            pl.BlockSpec((h_dim, p_dim), lambda i: (0, 0)),
            pl.BlockSpec((1, p_dim), lambda i: (0, 0)),
        ],
        out_specs=pl.BlockSpec((bn, p_dim), lambda i: (i, 0)),
        out_shape=jax.ShapeDtypeStruct((n, p_dim), F32),
    )(h, p0, p1, cntT, w2r, b2)


def kernel(x, edge_index, W1l, b1, W1r, W2l, b2, W2r):
    n, f = x.shape
    e = edge_index.shape[1]
    o = W2l.shape[1]
    p_dim = 128
    bn = 400

    nchunk = -(-e // (NW * CH))
    nchunk += nchunk % 2
    e_pad = nchunk * CH * NW
    src = jnp.concatenate(
        [edge_index[0], jnp.zeros((e_pad - e,), jnp.int32)]).reshape(
            NW, nchunk, 1, CH)
    dst = jnp.concatenate(
        [edge_index[1], jnp.full((e_pad - e,), n, jnp.int32)]).reshape(
            NW, nchunk, 1, CH)
    pad_chunk = jnp.concatenate(
        [jnp.zeros((NW, 1, 1, CH), jnp.int32),
         jnp.full((NW, 1, 1, CH), n, jnp.int32)], axis=2)
    edges = jnp.concatenate(
        [jnp.concatenate([src, dst], axis=2), pad_chunk], axis=1)

    rpt = ((n + NS * 8 - 1) // (NS * 8)) * 8
    zrow = jnp.zeros((rpt, f), F32)

    agg1 = _make_sc_agg(n, f, nchunk, with_count=True)
    acc0, acc1, cnt_part = agg1(x, edges, zrow)
    cntT = cnt_part.T

    w2l_p = jnp.pad(W2l, ((0, 0), (0, p_dim - o)))
    h, p = _tc_mid(x, acc0, acc1, cntT,
                   W1l, b1.reshape(1, -1), W1r, w2l_p, bn=bn)

    agg2 = _make_sc_agg(n, p_dim, nchunk, with_count=False)
    pacc0, pacc1 = agg2(p, edges, zrow)

    w2r_p = jnp.pad(W2r, ((0, 0), (0, p_dim - o)))
    b2_p = jnp.pad(b2, (0, p_dim - o)).reshape(1, -1)
    out = _tc_out(h, pacc0, pacc1, cntT, w2r_p, b2_p, bn=bn)
    return out[:, :o]
